# baseline (speedup 1.0000x reference)
import functools

import jax
import jax.numpy as jnp
from jax.experimental import pallas as pl
from jax.experimental.pallas import tpu as pltpu

_LANES = 128
_NEG = -3.0
_ROWS = 112


def _snorm_body(s_ref, out_ref, *, m_real):
    c = pl.program_id(1)
    s = s_ref[0]
    col = c * _LANES + jax.lax.broadcasted_iota(jnp.int32, s.shape, 1)
    s = jnp.where(col < m_real, s, 0.0)
    rs = 1.0 / (jnp.sqrt(jnp.sum(s * s, axis=0, keepdims=True)) + 1e-8)
    out_ref[...] = s * rs


def _top3_slice(inner, j, r0, r1, n_chunks, n_real_last, lane1):
    base = j * n_chunks

    def chunk(c):
        v = inner[r0:r1, (base + c) * _LANES:(base + c + 1) * _LANES]
        if c == n_chunks - 1 and n_real_last < _LANES:
            v = jnp.where(lane1 < n_real_last, v, _NEG)
        return v

    a = chunk(0)
    b = jnp.minimum(a, chunk(1))
    a = jnp.maximum(a, chunk(1))
    for c in range(2, n_chunks):
        v = chunk(c)
        a2 = jnp.maximum(a, v)
        t = jnp.minimum(a, v)
        b2 = jnp.maximum(b, t)
        u = jnp.minimum(b, t)
        if c == 2:
            cc = u
        else:
            cc = jnp.maximum(cc, u)
        a, b = a2, b2

    s3v = a + b + cc
    m1 = jnp.max(a, axis=1, keepdims=True)
    tri = jnp.max(jnp.where(a == m1, s3v, _NEG), axis=1, keepdims=True)
    eq1a, eq1b = a == m1, b == m1
    n1 = jnp.sum(jnp.where(eq1a, 1.0, 0.0) + jnp.where(eq1b, 1.0, 0.0),
                 axis=1, keepdims=True)
    a2x, b2x = jnp.where(eq1a, _NEG, a), jnp.where(eq1b, _NEG, b)
    m2 = jnp.max(jnp.maximum(a2x, b2x), axis=1, keepdims=True)
    eq2a, eq2b = a2x == m2, b2x == m2
    n2 = jnp.sum(jnp.where(eq2a, 1.0, 0.0) + jnp.where(eq2b, 1.0, 0.0),
                 axis=1, keepdims=True)
    a3x, b3x = jnp.where(eq2a, _NEG, a2x), jnp.where(eq2b, _NEG, b2x)
    m3 = jnp.max(jnp.maximum(a3x, b3x), axis=1, keepdims=True)
    t1 = jnp.minimum(n1, 3.0)
    t2 = jnp.clip(3.0 - n1, 0.0, n2)
    t3 = jnp.maximum(3.0 - n1 - n2, 0.0)
    top3 = jnp.maximum(m1 * t1 + m2 * t2 + m3 * t3, tri)
    return jnp.sum(top3, axis=0, keepdims=True)


def _knn_body(q_ref, s_ref, out_ref, *, n_way, m_real, m_pad):
    qb = q_ref[0]
    rq = 1.0 / (jnp.sqrt(jnp.sum(qb * qb, axis=0, keepdims=True)) + 1e-8)
    qn = qb * rq
    hw = qb.shape[1]
    n_chunks = m_pad // _LANES
    n_real_last = m_real - _LANES * (n_chunks - 1)
    lane1 = jax.lax.broadcasted_iota(jnp.int32, (1, _LANES), 1)
    inner = jax.lax.dot_general(
        qn, s_ref[...],
        dimension_numbers=(((0,), (0,)), ((), ())),
        preferred_element_type=jnp.float32,
    )
    bounds = list(range(0, hw, _ROWS)) + [hw]
    per_class = []
    for j in range(n_way):
        acc = None
        for r0, r1 in zip(bounds[:-1], bounds[1:]):
            part = _top3_slice(inner, j, r0, r1, n_chunks, n_real_last, lane1)
            acc = part if acc is None else acc + part
        per_class.append(acc)
    out = jnp.concatenate(per_class, axis=1)
    out_ref[...] = out[None]


def kernel(q, S, av_num):
    B, C, h, w = q.shape
    n_way, _, M = S.shape
    hw = h * w
    m_pad = ((M + _LANES - 1) // _LANES) * _LANES
    n_chunks = m_pad // _LANES
    qf = q.reshape(B, C, hw)

    Sn = pl.pallas_call(
        functools.partial(_snorm_body, m_real=M),
        grid=(n_way, n_chunks),
        in_specs=[pl.BlockSpec((1, C, _LANES), lambda j, c: (j, 0, c))],
        out_specs=pl.BlockSpec((C, _LANES), lambda j, c: (0, j * n_chunks + c)),
        out_shape=jax.ShapeDtypeStruct((C, n_way * m_pad), jnp.float32),
        compiler_params=pltpu.CompilerParams(
            dimension_semantics=("parallel", "parallel"),
        ),
    )(S)

    out = pl.pallas_call(
        functools.partial(_knn_body, n_way=n_way, m_real=M, m_pad=m_pad),
        grid=(B,),
        in_specs=[
            pl.BlockSpec((1, C, hw), lambda b: (b, 0, 0)),
            pl.BlockSpec((C, n_way * m_pad), lambda b: (0, 0)),
        ],
        out_specs=pl.BlockSpec((1, 1, n_way), lambda b: (b, 0, 0)),
        out_shape=jax.ShapeDtypeStruct((B, 1, n_way), jnp.float32),
        compiler_params=pltpu.CompilerParams(
            dimension_semantics=("parallel",),
        ),
    )(qf, Sn)
    out = out.reshape(B, n_way)
    return (out, out)


# speedup vs baseline: 1.0022x; 1.0022x over previous
import functools

import jax
import jax.numpy as jnp
from jax.experimental import pallas as pl
from jax.experimental.pallas import tpu as pltpu

_LANES = 128
_NEG = -3.0
_ROWS = 112


def _snorm_body(s_ref, out_ref, *, m_real):
    c = pl.program_id(1)
    s = s_ref[0]
    col = c * _LANES + jax.lax.broadcasted_iota(jnp.int32, s.shape, 1)
    s = jnp.where(col < m_real, s, 0.0)
    rs = 1.0 / (jnp.sqrt(jnp.sum(s * s, axis=0, keepdims=True)) + 1e-8)
    out_ref[...] = (s * rs)[None]


def _top3_sums(inner, n_chunks, n_real_last, lane1):
    def chunk(c):
        v = inner[:, c * _LANES:(c + 1) * _LANES]
        if c == n_chunks - 1 and n_real_last < _LANES:
            v = jnp.where(lane1 < n_real_last, v, _NEG)
        return v

    a = chunk(0)
    b = jnp.minimum(a, chunk(1))
    a = jnp.maximum(a, chunk(1))
    for c in range(2, n_chunks):
        v = chunk(c)
        a2 = jnp.maximum(a, v)
        t = jnp.minimum(a, v)
        b2 = jnp.maximum(b, t)
        u = jnp.minimum(b, t)
        if c == 2:
            cc = u
        else:
            cc = jnp.maximum(cc, u)
        a, b = a2, b2

    s3v = a + b + cc
    m1 = jnp.max(a, axis=1, keepdims=True)
    tri = jnp.max(jnp.where(a == m1, s3v, _NEG), axis=1, keepdims=True)
    eq1a, eq1b = a == m1, b == m1
    n1 = jnp.sum(jnp.where(eq1a, 1.0, 0.0) + jnp.where(eq1b, 1.0, 0.0),
                 axis=1, keepdims=True)
    a2x, b2x = jnp.where(eq1a, _NEG, a), jnp.where(eq1b, _NEG, b)
    m2 = jnp.max(jnp.maximum(a2x, b2x), axis=1, keepdims=True)
    eq2a, eq2b = a2x == m2, b2x == m2
    n2 = jnp.sum(jnp.where(eq2a, 1.0, 0.0) + jnp.where(eq2b, 1.0, 0.0),
                 axis=1, keepdims=True)
    a3x, b3x = jnp.where(eq2a, _NEG, a2x), jnp.where(eq2b, _NEG, b2x)
    m3 = jnp.max(jnp.maximum(a3x, b3x), axis=1, keepdims=True)
    t1 = jnp.minimum(n1, 3.0)
    t2 = jnp.clip(3.0 - n1, 0.0, n2)
    t3 = jnp.maximum(3.0 - n1 - n2, 0.0)
    top3 = jnp.maximum(m1 * t1 + m2 * t2 + m3 * t3, tri)
    return top3


def _knn_body(q_ref, s_ref, out_ref, *, n_way, m_real, m_pad):
    qb = q_ref[0]
    rq = 1.0 / (jnp.sqrt(jnp.sum(qb * qb, axis=0, keepdims=True)) + 1e-8)
    qn = qb * rq
    n_chunks = m_pad // _LANES
    n_real_last = m_real - _LANES * (n_chunks - 1)
    lane1 = jax.lax.broadcasted_iota(jnp.int32, (1, _LANES), 1)
    per_class = []
    for j in range(n_way):
        inner = jax.lax.dot_general(
            qn, s_ref[j],
            dimension_numbers=(((0,), (0,)), ((), ())),
            preferred_element_type=jnp.float32,
        )
        per_class.append(_top3_sums(inner, n_chunks, n_real_last, lane1))
    cat = jnp.concatenate(per_class, axis=1)
    out_ref[...] = jnp.sum(cat, axis=0, keepdims=True)[None]


def kernel(q, S, av_num):
    B, C, h, w = q.shape
    n_way, _, M = S.shape
    hw = h * w
    m_pad = ((M + _LANES - 1) // _LANES) * _LANES
    n_chunks = m_pad // _LANES
    qf = q.reshape(B, C, hw)

    Sn = pl.pallas_call(
        functools.partial(_snorm_body, m_real=M),
        grid=(n_way, n_chunks),
        in_specs=[pl.BlockSpec((1, C, _LANES), lambda j, c: (j, 0, c))],
        out_specs=pl.BlockSpec((1, C, _LANES), lambda j, c: (j, 0, c)),
        out_shape=jax.ShapeDtypeStruct((n_way, C, m_pad), jnp.float32),
        compiler_params=pltpu.CompilerParams(
            dimension_semantics=("parallel", "parallel"),
        ),
    )(S)

    out = pl.pallas_call(
        functools.partial(_knn_body, n_way=n_way, m_real=M, m_pad=m_pad),
        grid=(B,),
        in_specs=[
            pl.BlockSpec((1, C, hw), lambda b: (b, 0, 0)),
            pl.BlockSpec((n_way, C, m_pad), lambda b: (0, 0, 0)),
        ],
        out_specs=pl.BlockSpec((1, 1, n_way), lambda b: (b, 0, 0)),
        out_shape=jax.ShapeDtypeStruct((B, 1, n_way), jnp.float32),
        compiler_params=pltpu.CompilerParams(
            dimension_semantics=("parallel",),
        ),
    )(qf, Sn)
    out = out.reshape(B, n_way)
    return (out, out)


# speedup vs baseline: 1.0568x; 1.0545x over previous
import functools

import jax
import jax.numpy as jnp
from jax.experimental import pallas as pl
from jax.experimental.pallas import tpu as pltpu

_LANES = 128
_NEG = -3.0


def _snorm_body(s_ref, out_ref, *, m_real):
    c = pl.program_id(0)
    s = s_ref[...]
    col = c * _LANES + jax.lax.broadcasted_iota(jnp.int32, s.shape, 2)
    s = jnp.where(col < m_real, s, 0.0)
    rs = 1.0 / (jnp.sqrt(jnp.sum(s * s, axis=1, keepdims=True)) + 1e-8)
    out_ref[...] = s * rs


def _knn_body(q_ref, s_ref, out_ref, *, n_way, m_real):
    qb = q_ref[0]
    rq = 1.0 / (jnp.sqrt(jnp.sum(qb * qb, axis=0, keepdims=True)) + 1e-8)
    qn = qb * rq
    hw = qb.shape[1]
    m_pad = s_ref.shape[-1]
    n_chunks = m_pad // _LANES
    n_real_last = m_real - _LANES * (n_chunks - 1)
    lane = jax.lax.broadcasted_iota(jnp.int32, (hw, _LANES), 1)
    per_class = []
    for j in range(n_way):
        inner = jax.lax.dot_general(
            qn, s_ref[j],
            dimension_numbers=(((0,), (0,)), ((), ())),
            preferred_element_type=jnp.float32,
        )
        def chunk(c):
            v = inner[:, c * _LANES:(c + 1) * _LANES]
            if c == n_chunks - 1 and n_real_last < _LANES:
                v = jnp.where(lane < n_real_last, v, _NEG)
            return v
        a = chunk(0)
        b = jnp.minimum(a, chunk(1))
        a = jnp.maximum(a, chunk(1))
        for c in range(2, n_chunks):
            v = chunk(c)
            a2 = jnp.maximum(a, v)
            t = jnp.minimum(a, v)
            b2 = jnp.maximum(b, t)
            u = jnp.minimum(b, t)
            if c == 2:
                cc = u
            else:
                cc = jnp.maximum(cc, u)
            a, b = a2, b2
        cand = jnp.concatenate([a, b, cc], axis=1)
        m1 = jnp.max(cand, axis=1, keepdims=True)
        eq1 = cand == m1
        n1 = jnp.sum(eq1.astype(jnp.float32), axis=1, keepdims=True)
        s2 = jnp.where(eq1, _NEG, cand)
        m2 = jnp.max(s2, axis=1, keepdims=True)
        eq2 = s2 == m2
        n2 = jnp.sum(eq2.astype(jnp.float32), axis=1, keepdims=True)
        s3 = jnp.where(eq2, _NEG, s2)
        m3 = jnp.max(s3, axis=1, keepdims=True)
        t1 = jnp.minimum(n1, 3.0)
        t2 = jnp.clip(3.0 - n1, 0.0, n2)
        t3 = jnp.maximum(3.0 - n1 - n2, 0.0)
        per_class.append(m1 * t1 + m2 * t2 + m3 * t3)
    cat = jnp.concatenate(per_class, axis=1)
    out_ref[...] = jnp.sum(cat, axis=0, keepdims=True)[None]


def kernel(q, S, av_num):
    B, C, h, w = q.shape
    n_way, _, M = S.shape
    hw = h * w
    m_pad = ((M + _LANES - 1) // _LANES) * _LANES
    n_chunks = m_pad // _LANES
    qf = q.reshape(B, C, hw)

    Sn = pl.pallas_call(
        functools.partial(_snorm_body, m_real=M),
        grid=(n_chunks,),
        in_specs=[pl.BlockSpec((n_way, C, _LANES), lambda c: (0, 0, c))],
        out_specs=pl.BlockSpec((n_way, C, _LANES), lambda c: (0, 0, c)),
        out_shape=jax.ShapeDtypeStruct((n_way, C, m_pad), jnp.float32),
        compiler_params=pltpu.CompilerParams(
            dimension_semantics=("parallel",),
        ),
    )(S)

    out = pl.pallas_call(
        functools.partial(_knn_body, n_way=n_way, m_real=M),
        grid=(B,),
        in_specs=[
            pl.BlockSpec((1, C, hw), lambda b: (b, 0, 0)),
            pl.BlockSpec((n_way, C, m_pad), lambda b: (0, 0, 0)),
        ],
        out_specs=pl.BlockSpec((1, 1, n_way), lambda b: (b, 0, 0)),
        out_shape=jax.ShapeDtypeStruct((B, 1, n_way), jnp.float32),
        compiler_params=pltpu.CompilerParams(
            dimension_semantics=("parallel",),
        ),
    )(qf, Sn)
    out = out.reshape(B, n_way)
    return (out, out)
